# baseline (speedup 1.0000x reference)
import functools

import jax
import jax.numpy as jnp
from jax import lax
from jax.experimental import pallas as pl
from jax.experimental.pallas import tpu as pltpu
from jax.experimental.pallas import tpu_sc as plsc

_NBUF = 4


def kernel(inputs, table):
    B, T = inputs.shape
    V, D = table.shape
    idx = inputs.astype(jnp.int32)

    info = plsc.get_sparse_core_info()
    nw = info.num_cores * info.num_subcores
    b_per_w = B // nw

    mesh = plsc.VectorSubcoreMesh(core_axis_name="c", subcore_axis_name="s")

    @functools.partial(
        pl.kernel,
        out_type=jax.ShapeDtypeStruct((B, T, D), table.dtype),
        mesh=mesh,
        compiler_params=pltpu.CompilerParams(use_tc_tiling_on_sc=False),
        scratch_types=[
            pltpu.VMEM((b_per_w, T), jnp.int32),
            pltpu.VMEM((_NBUF, T, D), jnp.float32),
            pltpu.SemaphoreType.DMA,
            pltpu.SemaphoreType.DMA,
        ],
    )
    def gather_kernel(tab_hbm, idx_hbm, out_hbm, idx_v, rows_v, gsem, osem):
        wid = lax.axis_index("s") * info.num_cores + lax.axis_index("c")
        b0 = wid * b_per_w
        pltpu.sync_copy(idx_hbm.at[pl.ds(b0, b_per_w)], idx_v)

        for k in range(_NBUF):
            pltpu.make_async_copy(
                tab_hbm.at[idx_v.at[k]], rows_v.at[k], gsem
            ).start()

        @pl.loop(0, b_per_w, step=_NBUF)
        def _(j):
            for k in range(_NBUF):
                pltpu.make_async_copy(
                    tab_hbm.at[idx_v.at[j + k]], rows_v.at[k], gsem
                ).wait()
                pltpu.make_async_copy(
                    rows_v.at[k], out_hbm.at[b0 + j + k], osem
                ).start()

            for k in range(_NBUF):
                pltpu.make_async_copy(
                    rows_v.at[k], out_hbm.at[b0 + j + k], osem
                ).wait()

                @pl.when(j + _NBUF < b_per_w)
                def _():
                    pltpu.make_async_copy(
                        tab_hbm.at[idx_v.at[j + _NBUF + k]], rows_v.at[k], gsem
                    ).start()

    return gather_kernel(table, idx)
